# baseline (speedup 1.0000x reference)
import jax
import jax.numpy as jnp
from jax import lax
from jax.experimental import pallas as pl
from jax.experimental.pallas import tpu as pltpu
from jax.experimental.pallas import tpu_sc as plsc

_NUM_SEGMENTS = 10000
_N_VERTS = 320000
_D = 128
_LANES = 16
_NJ = _D // _LANES

_NC = 2
_NS = 16
_NW = _NC * _NS

_SEG_PW = 320
_OUT_PAD = _SEG_PW * _NW

_CHUNK = 256
_NCHUNKS = _N_VERTS // _CHUNK
_GROUP = 32
_BLK = 8


def _body(verts_hbm, ids_hbm, bounds_hbm, out_hbm,
          row_a, row_b, id_a, id_b, bounds_v, acc, sem_a, sem_b):
    wid = lax.axis_index("s") * _NC + lax.axis_index("c")
    s0 = wid * _SEG_PW

    pltpu.sync_copy(bounds_hbm, bounds_v)
    bv = bounds_v[wid, pl.ds(0, _LANES)]
    v0 = bv[0]
    v1 = bv[1]

    k0 = lax.div(v0, _CHUNK)
    k1 = lax.div(v1 + (_CHUNK - 1), _CHUNK)

    def start(k, rb, ib, sem):
        kc = jnp.clip(k, 0, _NCHUNKS - 1)
        base = kc * _CHUNK
        pltpu.async_copy(verts_hbm.at[pl.ds(base, _CHUNK)], rb, sem)
        pltpu.async_copy(ids_hbm.at[pl.ds(base, _CHUNK)], ib, sem)

    def wait(rb, ib, sem):
        pltpu.make_async_copy(verts_hbm.at[pl.ds(0, _CHUNK)], rb, sem).wait()
        pltpu.make_async_copy(ids_hbm.at[pl.ds(0, _CHUNK)], ib, sem).wait()

    start(k0, row_a, id_a, sem_a)

    neg = jnp.full((_LANES,), -jnp.inf, dtype=jnp.float32)

    def init_body(i, carry):
        for j in range(_NJ):
            acc[i, pl.ds(_LANES * j, _LANES)] = neg
        return carry

    lax.fori_loop(0, _SEG_PW + 1, init_body, 0)

    def process(k, rb, ib, carry):
        base = k * _CHUNK
        r_lo = jnp.maximum(v0 - base, 0)
        r_hi = jnp.minimum(v1 - base, _CHUNK)
        g_lo = lax.div(r_lo, _GROUP)
        g_hi = jnp.maximum(g_lo, lax.div(r_hi + (_GROUP - 1), _GROUP))

        def group_body(g, gc):
            pid, av = gc
            rbase = g * _GROUP
            idv = ib[pl.ds(rbase, _GROUP)]
            sids_raw = [idv[t] for t in range(_GROUP)]
            actives = [(rbase + t >= r_lo) & (rbase + t < r_hi)
                       for t in range(_GROUP)]
            sids = [jnp.where(actives[t], sids_raw[t], jnp.int32(-1))
                    for t in range(_GROUP)]
            keeps = [sids[0] == pid] + [sids[t] == sids[t - 1]
                                        for t in range(1, _GROUP)]
            lids = [jnp.where(actives[t], sids_raw[t] - s0, _SEG_PW)
                    for t in range(_GROUP)]
            def do_loads(t):
                r = rbase + t
                return [rb[r, pl.ds(_LANES * j, _LANES)]
                        for j in range(_NJ)]

            cur = do_loads(0)
            for t in range(_GROUP):
                nxt = do_loads(t + 1) if t + 1 < _GROUP else None
                masked = [jnp.where(keeps[t], av[j], neg)
                          for j in range(_NJ)]
                new = tuple(jnp.maximum(cur[j], masked[j])
                            for j in range(_NJ))
                for j in range(_NJ):
                    acc[lids[t], pl.ds(_LANES * j, _LANES)] = new[j]
                av = new
                cur = nxt
            return (sids[-1], av)

        return lax.fori_loop(g_lo, g_hi, group_body, carry)

    def pair_body(p, carry):
        k_a = k0 + 2 * p
        start(k_a + 1, row_b, id_b, sem_b)
        wait(row_a, id_a, sem_a)
        carry = process(k_a, row_a, id_a, carry)
        start(k_a + 2, row_a, id_a, sem_a)
        wait(row_b, id_b, sem_b)
        carry = process(k_a + 1, row_b, id_b, carry)
        return carry

    npairs = lax.div(k1 - k0 + 1, 2)
    lax.fori_loop(0, npairs, pair_body, (jnp.int32(-1), (neg,) * _NJ))
    wait(row_a, id_a, sem_a)

    pltpu.sync_copy(acc.at[pl.ds(0, _SEG_PW)], out_hbm.at[pl.ds(s0, _SEG_PW)])


@jax.jit
def _run(verts, ids, bounds):
    mesh = plsc.VectorSubcoreMesh(core_axis_name="c", subcore_axis_name="s")
    f = pl.kernel(
        _body,
        mesh=mesh,
        out_type=jax.ShapeDtypeStruct((_OUT_PAD, _D), jnp.float32),
        scratch_types=[
            pltpu.VMEM((_CHUNK, _D), jnp.float32),
            pltpu.VMEM((_CHUNK, _D), jnp.float32),
            pltpu.VMEM((_CHUNK,), jnp.int32),
            pltpu.VMEM((_CHUNK,), jnp.int32),
            pltpu.VMEM((_NW, _LANES), jnp.int32),
            pltpu.VMEM((_SEG_PW + 1, _D), jnp.float32),
            pltpu.SemaphoreType.DMA,
            pltpu.SemaphoreType.DMA,
        ],
    )
    return f(verts, ids, bounds)


def kernel(verts, verts_idx):
    ids = verts_idx.astype(jnp.int32)
    seg_starts = jnp.arange(_NW + 1, dtype=jnp.int32) * _SEG_PW
    vb = jnp.searchsorted(ids, seg_starts, side="left",
                          method="compare_all").astype(jnp.int32)
    vb2 = jnp.zeros((_NW, _LANES), jnp.int32)
    vb2 = vb2.at[:, 0].set(vb[:-1]).at[:, 1].set(vb[1:])
    out = _run(verts, ids, vb2)
    return out[:_NUM_SEGMENTS]


# speedup vs baseline: 1.0280x; 1.0280x over previous
import jax
import jax.numpy as jnp
from jax import lax
from jax.experimental import pallas as pl
from jax.experimental.pallas import tpu as pltpu
from jax.experimental.pallas import tpu_sc as plsc

_NUM_SEGMENTS = 10000
_N_VERTS = 320000
_D = 128
_LANES = 16
_NJ = _D // _LANES

_NC = 2
_NS = 16
_NW = _NC * _NS

_SEG_PW = 320
_OUT_PAD = _SEG_PW * _NW

_CHUNK = 320
_NCHUNKS = _N_VERTS // _CHUNK
_GROUP = 16
_BLK = 8


def _body(verts_hbm, ids_hbm, bounds_hbm, out_hbm,
          row_a, row_b, id_a, id_b, bounds_v, acc, sem_a, sem_b):
    wid = lax.axis_index("s") * _NC + lax.axis_index("c")
    s0 = wid * _SEG_PW

    pltpu.sync_copy(bounds_hbm, bounds_v)
    bv = bounds_v[wid, pl.ds(0, _LANES)]
    v0 = bv[0]
    v1 = bv[1]

    k0 = lax.div(v0, _CHUNK)
    k1 = lax.div(v1 + (_CHUNK - 1), _CHUNK)

    def start(k, rb, ib, sem):
        kc = jnp.clip(k, 0, _NCHUNKS - 1)
        base = kc * _CHUNK
        pltpu.async_copy(verts_hbm.at[pl.ds(base, _CHUNK)], rb, sem)
        pltpu.async_copy(ids_hbm.at[pl.ds(base, _CHUNK)], ib, sem)

    def wait(rb, ib, sem):
        pltpu.make_async_copy(verts_hbm.at[pl.ds(0, _CHUNK)], rb, sem).wait()
        pltpu.make_async_copy(ids_hbm.at[pl.ds(0, _CHUNK)], ib, sem).wait()

    start(k0, row_a, id_a, sem_a)

    neg = jnp.full((_LANES,), -jnp.inf, dtype=jnp.float32)

    def init_body(i, carry):
        for j in range(_NJ):
            acc[i, pl.ds(_LANES * j, _LANES)] = neg
        return carry

    lax.fori_loop(0, _SEG_PW + 1, init_body, 0)

    def process(k, rb, ib, carry):
        base = k * _CHUNK
        r_lo = jnp.maximum(v0 - base, 0)
        r_hi = jnp.minimum(v1 - base, _CHUNK)
        g_lo = lax.div(r_lo, _GROUP)
        g_hi = jnp.maximum(g_lo, lax.div(r_hi + (_GROUP - 1), _GROUP))

        def group_body(g, gc):
            pid, av = gc
            rbase = g * _GROUP
            idv = ib[pl.ds(rbase, _GROUP)]
            sids_raw = [idv[t] for t in range(_GROUP)]
            actives = [(rbase + t >= r_lo) & (rbase + t < r_hi)
                       for t in range(_GROUP)]
            sids = [jnp.where(actives[t], sids_raw[t], jnp.int32(-1))
                    for t in range(_GROUP)]
            keeps = [sids[0] == pid] + [sids[t] == sids[t - 1]
                                        for t in range(1, _GROUP)]
            lids = [jnp.where(actives[t], sids_raw[t] - s0, _SEG_PW)
                    for t in range(_GROUP)]
            def do_loads(t):
                r = rbase + t
                return [rb[r, pl.ds(_LANES * j, _LANES)]
                        for j in range(_NJ)]

            cur = do_loads(0)
            for t in range(_GROUP):
                nxt = do_loads(t + 1) if t + 1 < _GROUP else None
                masked = [jnp.where(keeps[t], av[j], neg)
                          for j in range(_NJ)]
                new = tuple(jnp.maximum(cur[j], masked[j])
                            for j in range(_NJ))
                for j in range(_NJ):
                    acc[lids[t], pl.ds(_LANES * j, _LANES)] = new[j]
                av = new
                cur = nxt
            return (sids[-1], av)

        return lax.fori_loop(g_lo, g_hi, group_body, carry)

    def pair_body(p, carry):
        k_a = k0 + 2 * p
        start(k_a + 1, row_b, id_b, sem_b)
        wait(row_a, id_a, sem_a)
        carry = process(k_a, row_a, id_a, carry)
        start(k_a + 2, row_a, id_a, sem_a)
        wait(row_b, id_b, sem_b)
        carry = process(k_a + 1, row_b, id_b, carry)
        return carry

    npairs = lax.div(k1 - k0 + 1, 2)
    lax.fori_loop(0, npairs, pair_body, (jnp.int32(-1), (neg,) * _NJ))
    wait(row_a, id_a, sem_a)

    pltpu.sync_copy(acc.at[pl.ds(0, _SEG_PW)], out_hbm.at[pl.ds(s0, _SEG_PW)])


@jax.jit
def _run(verts, ids, bounds):
    mesh = plsc.VectorSubcoreMesh(core_axis_name="c", subcore_axis_name="s")
    f = pl.kernel(
        _body,
        mesh=mesh,
        out_type=jax.ShapeDtypeStruct((_OUT_PAD, _D), jnp.float32),
        scratch_types=[
            pltpu.VMEM((_CHUNK, _D), jnp.float32),
            pltpu.VMEM((_CHUNK, _D), jnp.float32),
            pltpu.VMEM((_CHUNK,), jnp.int32),
            pltpu.VMEM((_CHUNK,), jnp.int32),
            pltpu.VMEM((_NW, _LANES), jnp.int32),
            pltpu.VMEM((_SEG_PW + 1, _D), jnp.float32),
            pltpu.SemaphoreType.DMA,
            pltpu.SemaphoreType.DMA,
        ],
    )
    return f(verts, ids, bounds)


def kernel(verts, verts_idx):
    ids = verts_idx.astype(jnp.int32)
    seg_starts = jnp.arange(_NW + 1, dtype=jnp.int32) * _SEG_PW
    vb = jnp.searchsorted(ids, seg_starts, side="left",
                          method="compare_all").astype(jnp.int32)
    vb2 = jnp.zeros((_NW, _LANES), jnp.int32)
    vb2 = vb2.at[:, 0].set(vb[:-1]).at[:, 1].set(vb[1:])
    out = _run(verts, ids, vb2)
    return out[:_NUM_SEGMENTS]


# speedup vs baseline: 1.0538x; 1.0251x over previous
import jax
import jax.numpy as jnp
from jax import lax
from jax.experimental import pallas as pl
from jax.experimental.pallas import tpu as pltpu
from jax.experimental.pallas import tpu_sc as plsc

_NUM_SEGMENTS = 10000
_N_VERTS = 320000
_D = 128
_LANES = 16
_NJ = _D // _LANES

_NC = 2
_NS = 16
_NW = _NC * _NS

_SEG_PW = 320
_OUT_PAD = _SEG_PW * _NW

_CHUNK = 256
_NCHUNKS = _N_VERTS // _CHUNK
_GROUP = 16
_BLK = 8


def _body(verts_hbm, ids_hbm, bounds_hbm, out_hbm,
          row_a, row_b, id_a, id_b, bounds_v, acc, sem_a, sem_b):
    wid = lax.axis_index("s") * _NC + lax.axis_index("c")
    s0 = wid * _SEG_PW

    pltpu.sync_copy(bounds_hbm, bounds_v)
    bv = bounds_v[wid, pl.ds(0, _LANES)]
    v0 = bv[0]
    v1 = bv[1]

    k0 = lax.div(v0, _CHUNK)
    k1 = lax.div(v1 + (_CHUNK - 1), _CHUNK)

    def start(k, rb, ib, sem):
        kc = jnp.clip(k, 0, _NCHUNKS - 1)
        base = kc * _CHUNK
        pltpu.async_copy(verts_hbm.at[pl.ds(base, _CHUNK)], rb, sem)
        pltpu.async_copy(ids_hbm.at[pl.ds(base, _CHUNK)], ib, sem)

    def wait(rb, ib, sem):
        pltpu.make_async_copy(verts_hbm.at[pl.ds(0, _CHUNK)], rb, sem).wait()
        pltpu.make_async_copy(ids_hbm.at[pl.ds(0, _CHUNK)], ib, sem).wait()

    start(k0, row_a, id_a, sem_a)

    neg = jnp.full((_LANES,), -jnp.inf, dtype=jnp.float32)

    def init_body(i, carry):
        for j in range(_NJ):
            acc[i, pl.ds(_LANES * j, _LANES)] = neg
        return carry

    lax.fori_loop(0, _SEG_PW + 1, init_body, 0)

    def process(k, rb, ib, carry):
        base = k * _CHUNK
        r_lo = jnp.maximum(v0 - base, 0)
        r_hi = jnp.minimum(v1 - base, _CHUNK)
        g_lo = lax.div(r_lo, _GROUP)
        g_hi = jnp.maximum(g_lo, lax.div(r_hi + (_GROUP - 1), _GROUP))

        def group_body(g, gc):
            pid, av = gc
            rbase = g * _GROUP
            idv = ib[pl.ds(rbase, _GROUP)]
            sids_raw = [idv[t] for t in range(_GROUP)]
            actives = [(rbase + t >= r_lo) & (rbase + t < r_hi)
                       for t in range(_GROUP)]
            sids = [jnp.where(actives[t], sids_raw[t], jnp.int32(-1))
                    for t in range(_GROUP)]
            keeps = [sids[0] == pid] + [sids[t] == sids[t - 1]
                                        for t in range(1, _GROUP)]
            lids = [jnp.where(actives[t], sids_raw[t] - s0, _SEG_PW)
                    for t in range(_GROUP)]
            def do_loads(t):
                r = rbase + t
                return [rb[r, pl.ds(_LANES * j, _LANES)]
                        for j in range(_NJ)]

            pending = [do_loads(0), do_loads(1)]
            for t in range(_GROUP):
                if t + 2 < _GROUP:
                    pending.append(do_loads(t + 2))
                cur = pending.pop(0)
                masked = [jnp.where(keeps[t], av[j], neg)
                          for j in range(_NJ)]
                new = tuple(jnp.maximum(cur[j], masked[j])
                            for j in range(_NJ))
                for j in range(_NJ):
                    acc[lids[t], pl.ds(_LANES * j, _LANES)] = new[j]
                av = new
            return (sids[-1], av)

        return lax.fori_loop(g_lo, g_hi, group_body, carry)

    def pair_body(p, carry):
        k_a = k0 + 2 * p
        start(k_a + 1, row_b, id_b, sem_b)
        wait(row_a, id_a, sem_a)
        carry = process(k_a, row_a, id_a, carry)
        start(k_a + 2, row_a, id_a, sem_a)
        wait(row_b, id_b, sem_b)
        carry = process(k_a + 1, row_b, id_b, carry)
        return carry

    npairs = lax.div(k1 - k0 + 1, 2)
    lax.fori_loop(0, npairs, pair_body, (jnp.int32(-1), (neg,) * _NJ))
    wait(row_a, id_a, sem_a)

    pltpu.sync_copy(acc.at[pl.ds(0, _SEG_PW)], out_hbm.at[pl.ds(s0, _SEG_PW)])


@jax.jit
def _run(verts, ids, bounds):
    mesh = plsc.VectorSubcoreMesh(core_axis_name="c", subcore_axis_name="s")
    f = pl.kernel(
        _body,
        mesh=mesh,
        out_type=jax.ShapeDtypeStruct((_OUT_PAD, _D), jnp.float32),
        scratch_types=[
            pltpu.VMEM((_CHUNK, _D), jnp.float32),
            pltpu.VMEM((_CHUNK, _D), jnp.float32),
            pltpu.VMEM((_CHUNK,), jnp.int32),
            pltpu.VMEM((_CHUNK,), jnp.int32),
            pltpu.VMEM((_NW, _LANES), jnp.int32),
            pltpu.VMEM((_SEG_PW + 1, _D), jnp.float32),
            pltpu.SemaphoreType.DMA,
            pltpu.SemaphoreType.DMA,
        ],
    )
    return f(verts, ids, bounds)


def kernel(verts, verts_idx):
    ids = verts_idx.astype(jnp.int32)
    seg_starts = jnp.arange(_NW + 1, dtype=jnp.int32) * _SEG_PW
    vb = jnp.searchsorted(ids, seg_starts, side="left",
                          method="compare_all").astype(jnp.int32)
    vb2 = jnp.zeros((_NW, _LANES), jnp.int32)
    vb2 = vb2.at[:, 0].set(vb[:-1]).at[:, 1].set(vb[1:])
    out = _run(verts, ids, vb2)
    return out[:_NUM_SEGMENTS]
